# Initial kernel scaffold; baseline (speedup 1.0000x reference)
import jax
import jax.numpy as jnp
from jax.experimental import pallas as pl


def kernel(v_h, edge_list, edge_weights, W_g, b_g, W1, b1, W2, b2):
    raise NotImplementedError("write your pallas kernel here")


# speedup vs baseline: 6.2392x; 6.2392x over previous
import functools

import jax
import jax.numpy as jnp
from jax import lax
from jax.experimental import pallas as pl
from jax.experimental.pallas import tpu as pltpu
from jax.experimental.pallas import tpu_sc as plsc

N = 10000
E = 320000
D = 256
H = 256
NP = 10240
NC = 2
NS = 16
CHUNK = 80
RB = 512
TE = 1024


def _deg_body(dst_hbm, ew_hbm, deg_hbm, idx_v, val_v, zero_v, deg_sp):
    c = lax.axis_index("c")
    s = lax.axis_index("s")

    @pl.when(c == 0)
    def _():
        for j in range(640 // 16):
            zero_v[pl.ds(j * 16, 16)] = jnp.zeros((16,), jnp.float32)
        pltpu.sync_copy(zero_v, deg_sp.at[pl.ds(s * 640, 640)])
        plsc.subcore_barrier()

        epc = E // NS
        base = s * epc

        def chunk(t, carry):
            off = base + t * CHUNK
            pltpu.sync_copy(dst_hbm.at[pl.ds(off, CHUNK)], idx_v)
            pltpu.sync_copy(ew_hbm.at[pl.ds(off, CHUNK)], val_v)
            pltpu.sync_copy(val_v, deg_sp.at[idx_v], add=True)
            return carry

        lax.fori_loop(0, epc // CHUNK, chunk, 0)
        plsc.subcore_barrier()
        pltpu.sync_copy(deg_sp.at[pl.ds(s * 640, 640)],
                        deg_hbm.at[pl.ds(s * 640, 640)])


def _deg_call(dst, ew):
    mesh = plsc.VectorSubcoreMesh(core_axis_name="c", subcore_axis_name="s")
    fn = pl.kernel(
        _deg_body,
        out_type=jax.ShapeDtypeStruct((NP,), jnp.float32),
        mesh=mesh,
        scratch_types=[
            pltpu.VMEM((CHUNK,), jnp.int32),
            pltpu.VMEM((CHUNK,), jnp.float32),
            pltpu.VMEM((640,), jnp.float32),
            pltpu.VMEM_SHARED((NP,), jnp.float32),
        ],
    )
    return fn(dst, ew)


def _msg_body(yflat_hbm, src_hbm, dst_hbm, ew_hbm, zout_hbm,
              src_v, dst_v, ew_v, rows_v, sem, z_sp):
    c = lax.axis_index("c")
    s = lax.axis_index("s")
    rows_per_tile = NP // NS

    pltpu.sync_copy(yflat_hbm.at[pl.ds(c * NP + s * rows_per_tile, rows_per_tile)],
                    z_sp.at[pl.ds(s * rows_per_tile, rows_per_tile)])
    plsc.subcore_barrier()

    epc = E // NS
    base = s * epc
    cN = c * NP

    def chunk(t, carry):
        off = base + t * CHUNK
        pltpu.sync_copy(src_hbm.at[pl.ds(off, CHUNK)], src_v)
        pltpu.sync_copy(dst_hbm.at[pl.ds(off, CHUNK)], dst_v)
        pltpu.sync_copy(ew_hbm.at[pl.ds(off, CHUNK)], ew_v)
        for j in range(CHUNK // 16):
            src_v[pl.ds(j * 16, 16)] = src_v[pl.ds(j * 16, 16)] + cN
        pltpu.async_copy(yflat_hbm.at[src_v], rows_v, sem).wait()
        for g in range(CHUNK // 16):
            grp = ew_v[pl.ds(g * 16, 16)]
            for l in range(16):
                i = g * 16 + l
                ew_s = grp[l]
                for j in range(8):
                    rows_v[i, pl.ds(j * 16, 16)] = rows_v[i, pl.ds(j * 16, 16)] * ew_s
        pltpu.sync_copy(rows_v, z_sp.at[dst_v], add=True)
        return carry

    lax.fori_loop(0, epc // CHUNK, chunk, 0)
    plsc.subcore_barrier()
    pltpu.sync_copy(z_sp.at[pl.ds(s * rows_per_tile, rows_per_tile)],
                    zout_hbm.at[pl.ds(c * NP + s * rows_per_tile, rows_per_tile)])


def _msg_call(yflat, src, dst, ew):
    mesh = plsc.VectorSubcoreMesh(core_axis_name="c", subcore_axis_name="s")
    fn = pl.kernel(
        _msg_body,
        out_type=jax.ShapeDtypeStruct((2 * NP, 128), jnp.float32),
        mesh=mesh,
        scratch_types=[
            pltpu.VMEM((CHUNK,), jnp.int32),
            pltpu.VMEM((CHUNK,), jnp.int32),
            pltpu.VMEM((CHUNK,), jnp.float32),
            pltpu.VMEM((CHUNK, 128), jnp.float32),
            pltpu.SemaphoreType.DMA,
            pltpu.VMEM_SHARED((NP, 128), jnp.float32),
        ],
    )
    return fn(yflat, src, dst, ew)


def _y_body(vh_ref, w_ref, deg_ref, out_ref):
    h = pl.program_id(0)
    xl = jnp.dot(vh_ref[...], w_ref[...], preferred_element_type=jnp.float32)
    dis = lax.rsqrt(1.0 + deg_ref[...][:, 0])
    y = xl * dis[:, None]
    out_ref[...] = jnp.where(h == 0, y[:, :128], y[:, 128:])


def _y_call(v_h, W_g, deg2d):
    nb = NP // RB
    return pl.pallas_call(
        _y_body,
        grid=(2, nb),
        in_specs=[
            pl.BlockSpec((RB, D), lambda h, i: (i, 0)),
            pl.BlockSpec((D, H), lambda h, i: (0, 0)),
            pl.BlockSpec((RB, 1), lambda h, i: (i, 0)),
        ],
        out_specs=pl.BlockSpec((RB, 128), lambda h, i: (h * (NP // RB) + i, 0)),
        out_shape=jax.ShapeDtypeStruct((2 * NP, 128), jnp.float32),
    )(v_h, W_g, deg2d)


def _embed_body(z0_ref, z1_ref, deg_ref, bg_ref, w1_ref, b1_ref, w2_ref, b2_ref,
                de1_ref, de2_ref):
    z = jnp.concatenate([z0_ref[...], z1_ref[...]], axis=1)
    dis = lax.rsqrt(1.0 + deg_ref[...][:, 0])
    df = z * dis[:, None] + bg_ref[...]
    de1_ref[...] = jnp.tanh(
        jnp.dot(df, w1_ref[...], preferred_element_type=jnp.float32) + b1_ref[...])
    de2_ref[...] = jnp.tanh(
        jnp.dot(df, w2_ref[...], preferred_element_type=jnp.float32) + b2_ref[...])


def _embed_call(zflat, deg2d, b_g, W1, b1, W2, b2):
    nb = NP // RB
    out_sds = jax.ShapeDtypeStruct((N, H), jnp.float32)
    return pl.pallas_call(
        _embed_body,
        grid=(nb,),
        in_specs=[
            pl.BlockSpec((RB, 128), lambda i: (i, 0)),
            pl.BlockSpec((RB, 128), lambda i, _nb=nb: (_nb + i, 0)),
            pl.BlockSpec((RB, 1), lambda i: (i, 0)),
            pl.BlockSpec((1, H), lambda i: (0, 0)),
            pl.BlockSpec((H, H), lambda i: (0, 0)),
            pl.BlockSpec((1, H), lambda i: (0, 0)),
            pl.BlockSpec((H, H), lambda i: (0, 0)),
            pl.BlockSpec((1, H), lambda i: (0, 0)),
        ],
        out_specs=[
            pl.BlockSpec((RB, H), lambda i: (i, 0)),
            pl.BlockSpec((RB, H), lambda i: (i, 0)),
        ],
        out_shape=[out_sds, out_sds],
    )(zflat, zflat, deg2d, b_g.reshape(1, H), W1, b1.reshape(1, H),
      W2, b2.reshape(1, H))


def _sim_body(a_i, b_j, b_i, a_j, out_ref):
    dn = (((1,), (1,)), ((), ()))
    s1 = lax.dot_general(a_i[...], b_j[...], dn, preferred_element_type=jnp.float32)
    s2 = lax.dot_general(b_i[...], a_j[...], dn, preferred_element_type=jnp.float32)
    out_ref[...] = jnp.maximum(jnp.tanh(s1 - s2), 0.0)


def _sim_call(de1, de2):
    nb = pl.cdiv(N, TE)
    return pl.pallas_call(
        _sim_body,
        grid=(nb, nb),
        in_specs=[
            pl.BlockSpec((TE, H), lambda i, j: (i, 0)),
            pl.BlockSpec((TE, H), lambda i, j: (j, 0)),
            pl.BlockSpec((TE, H), lambda i, j: (i, 0)),
            pl.BlockSpec((TE, H), lambda i, j: (j, 0)),
        ],
        out_specs=pl.BlockSpec((TE, TE), lambda i, j: (i, j)),
        out_shape=jax.ShapeDtypeStruct((N, N), jnp.float32),
    )(de1, de2, de2, de1)


def kernel(v_h, edge_list, edge_weights, W_g, b_g, W1, b1, W2, b2):
    src = edge_list[0]
    dst = edge_list[1]
    deg = _deg_call(dst, edge_weights)
    deg2d = deg.reshape(NP, 1)
    yflat = _y_call(v_h, W_g, deg2d)
    zflat = _msg_call(yflat, src, dst, edge_weights)
    de1, de2 = _embed_call(zflat, deg2d, b_g, W1, b1, W2, b2)
    return _sim_call(de1, de2)
